# baseline (speedup 1.0000x reference)
import functools

import jax
import jax.numpy as jnp
from jax import lax
from jax.experimental import pallas as pl
from jax.experimental.pallas import tpu as pltpu
from jax.experimental.pallas import tpu_sc as plsc

VOCAB = 1000000
D = 64
B = 16384
NC = 2
NS = 16
NW = NC * NS
BPW = B // NW
CH = 128
NCH = BPW // CH
LANES = 16

@functools.cache
def _build_encoder():
    mesh = plsc.VectorSubcoreMesh(core_axis_name="c", subcore_axis_name="s")

    @functools.partial(
        pl.kernel,
        mesh=mesh,
        out_type=jax.ShapeDtypeStruct((B, D), jnp.float32),
        scratch_types=[
            pltpu.VMEM((NCH, CH), jnp.int32),
            pltpu.VMEM((BPW, D), jnp.float32),
            pltpu.VMEM((BPW, D), jnp.float32),
            pltpu.SemaphoreType.DMA,
            pltpu.SemaphoreType.DMA,
        ],
        compiler_params=pltpu.CompilerParams(use_tc_tiling_on_sc=False),
    )
    def _encode(idx_hbm, item_hbm, ent_hbm, out_hbm, idx_v, a_v, b_v,
                sem_a, sem_b):
        wid = lax.axis_index("s") * NC + lax.axis_index("c")
        base = wid * BPW

        pltpu.sync_copy(idx_hbm.at[pl.ds(wid * NCH, NCH)], idx_v)

        copies = []
        for j in range(NCH):
            dst = pl.ds(j * CH, CH)
            copies.append(
                pltpu.async_copy(item_hbm.at[idx_v.at[j]], a_v.at[dst], sem_a))
            copies.append(
                pltpu.async_copy(ent_hbm.at[idx_v.at[j]], b_v.at[dst], sem_b))
        for c in copies:
            c.wait()

        def body(i, carry):
            for jj in range(D // LANES):
                s = pl.ds(jj * LANES, LANES)
                a_v[i, s] = a_v[i, s] + b_v[i, s]
            return carry

        lax.fori_loop(0, BPW, body, 0)

        pltpu.sync_copy(a_v, out_hbm.at[pl.ds(base, BPW)])

    return _encode


def kernel(batch_data, item_table, ent_table):
    idx2d = batch_data.reshape(NW * NCH, CH)
    return _build_encoder()(idx2d, item_table, ent_table)
